# baseline (speedup 1.0000x reference)
import functools

import jax
import jax.numpy as jnp
from jax import lax
from jax.experimental import pallas as pl
from jax.experimental.pallas import tpu as pltpu
from jax.experimental.pallas import tpu_sc as plsc

NC = 2
NS = 16
LANES = 16

KB = 80
KJ = 5
K = KB * KJ


def _prep(v, ew2, D):
  db = 5120
  nblk = -(D // -db)
  R, C = ew2.shape

  def body(v_ref, ew_ref, ot_ref, ow_ref):
    ot_ref[...] = v_ref[...].T[None]
    x = ew_ref[...]
    ow_ref[...] = jnp.maximum(x, 0.0) + jnp.log(1.0 + jnp.exp(-jnp.abs(x)))

  return pl.pallas_call(
      body,
      grid=(2, nblk),
      in_specs=[
          pl.BlockSpec((32, db), lambda c, i: (c, i)),
          pl.BlockSpec((R, db), lambda c, i: (0, i)),
      ],
      out_specs=[
          pl.BlockSpec((1, db, 32), lambda c, i: (c, i, 0)),
          pl.BlockSpec((R, db), lambda c, i: (0, i)),
      ],
      out_shape=[
          jax.ShapeDtypeStruct((2, D, 32), jnp.float32),
          jax.ShapeDtypeStruct((R, C), jnp.float32),
      ],
  )(v, ew2)


def _combine(v, avt3, degp, D):
  db = 5120
  nblk = -(D // -db)

  def body(v_ref, a_ref, d_ref, o_ref):
    deg = jnp.sum(d_ref[...], axis=0)
    av = a_ref[0].T
    o_ref[...] = deg[None, :] * v_ref[...] - av

  return pl.pallas_call(
      body,
      grid=(2, nblk),
      in_specs=[
          pl.BlockSpec((32, db), lambda c, i: (c, i)),
          pl.BlockSpec((1, db, 32), lambda c, i: (c, i, 0)),
          pl.BlockSpec((NC, db), lambda c, i: (0, i)),
      ],
      out_specs=pl.BlockSpec((32, db), lambda c, i: (c, i)),
      out_shape=jax.ShapeDtypeStruct((64, D), jnp.float32),
  )(v, avt3, degp)


def _make_sc_scatter(D, E):
  EDIR = 2 * E
  EPT = EDIR // NS
  NCHUNK = EPT // K
  RPT = -(D // -NS) + 7 - (-(D // -NS) + 7) % 8
  RLAST = D - (NS - 1) * RPT
  mesh = plsc.VectorSubcoreMesh(core_axis_name="c", subcore_axis_name="s")

  def _segs(total):
    out, o = [], 0
    while o < total:
      s = min(KB, total - o)
      out.append((o, s))
      o += s
    return out

  SEGS_FULL = _segs(RPT)
  SEGS_LAST = _segs(RLAST)

  @functools.partial(
      pl.kernel,
      out_type=jax.ShapeDtypeStruct((2 * D, 32), jnp.float32),
      mesh=mesh,
      compiler_params=pltpu.CompilerParams(use_tc_tiling_on_sc=False),
      scratch_types=[
          pltpu.VMEM((3, KJ, KB), jnp.int32),
          pltpu.VMEM((3, KJ, KB), jnp.int32),
          pltpu.VMEM((3, K), jnp.float32),
          pltpu.VMEM((2, KJ, KB, 32), jnp.float32),
          pltpu.VMEM_SHARED((D, 32), jnp.float32),
          pltpu.SemaphoreType.DMA,
          pltpu.SemaphoreType.DMA,
          pltpu.SemaphoreType.DMA,
          pltpu.SemaphoreType.DMA,
      ],
  )
  def sc_scatter(table, wvec, dsta, srca, avt,
                 dst_v, src_v, w_v, rows_v, acc,
                 lsem, gsem0, gsem1, ssem):
    cid = lax.axis_index("c")
    sid = lax.axis_index("s")
    zero16 = jnp.zeros((LANES,), jnp.float32)

    def _z2(r, _):
      rows_v[0, 0, r, pl.ds(0, LANES)] = zero16
      rows_v[0, 0, r, pl.ds(LANES, LANES)] = zero16
      return 0
    lax.fori_loop(0, KB, _z2, 0)

    def _zero_acc(segs):
      base = sid * RPT
      for (o, s) in segs:
        pltpu.sync_copy(rows_v.at[0, 0, pl.ds(0, s)],
                        acc.at[pl.ds(base + o, s)])

    @pl.when(sid < NS - 1)
    def _():
      _zero_acc(SEGS_FULL)

    @pl.when(sid == NS - 1)
    def _():
      _zero_acc(SEGS_LAST)

    plsc.subcore_barrier()

    coff16 = jnp.full((LANES,), cid * D, jnp.int32)
    ebase = sid * EPT
    wbase = ebase - jnp.where(sid >= NS // 2, E, 0)

    def idx_issue(ci):
      ph = lax.rem(ci, 3)
      off = ebase + ci * K
      for j in range(KJ):
        pltpu.async_copy(dsta.at[pl.ds(off + j * KB, KB)], dst_v.at[ph, j],
                         lsem)
        pltpu.async_copy(srca.at[pl.ds(off + j * KB, KB)], src_v.at[ph, j],
                         lsem)
      pltpu.async_copy(wvec.at[pl.ds(wbase + ci * K, K)], w_v.at[ph], lsem)

    def idx_wait(ci):
      ph = lax.rem(ci, 3)
      for j in range(KJ):
        pltpu.make_async_copy(dsta.at[pl.ds(0, KB)], dst_v.at[ph, j],
                              lsem).wait()
        pltpu.make_async_copy(srca.at[pl.ds(0, KB)], src_v.at[ph, j],
                              lsem).wait()
      pltpu.make_async_copy(wvec.at[pl.ds(0, K)], w_v.at[ph], lsem).wait()

    def adjust(ci):
      ph = lax.rem(ci, 3)
      for j in range(KJ):
        for g in range(KB // LANES):
          sl = pl.ds(g * LANES, LANES)
          src_v[ph, j, sl] = src_v[ph, j, sl] + coff16

    def gather_issue(ci):
      ph = lax.rem(ci, 3)
      for p2, sem in ((0, gsem0), (1, gsem1)):
        @pl.when(lax.rem(ci, 2) == p2)
        def _(p2=p2, sem=sem):
          for j in range(KJ):
            pltpu.async_copy(table.at[src_v.at[ph, j]], rows_v.at[p2, j], sem)

    def gather_wait(ci):
      for p2, sem in ((0, gsem0), (1, gsem1)):
        @pl.when(lax.rem(ci, 2) == p2)
        def _(p2=p2, sem=sem):
          for j in range(KJ):
            pltpu.make_async_copy(table.at[pl.ds(0, KB)], rows_v.at[p2, j],
                                  sem).wait()

    def scale(ci):
      p2 = lax.rem(ci, 2)
      ph = lax.rem(ci, 3)
      for j in range(KJ):
        def body(g, _):
          w16 = w_v[ph, pl.ds(j * KB + g * LANES, LANES)]
          for i in range(LANES):
            r = g * LANES + i
            ws = jnp.full((LANES,), w16[i], jnp.float32)
            rows_v[p2, j, r, pl.ds(0, LANES)] = (
                rows_v[p2, j, r, pl.ds(0, LANES)] * ws)
            rows_v[p2, j, r, pl.ds(LANES, LANES)] = (
                rows_v[p2, j, r, pl.ds(LANES, LANES)] * ws)
          return 0
        lax.fori_loop(0, KB // LANES, body, 0)

    def scatter_issue(ci):
      p2 = lax.rem(ci, 2)
      ph = lax.rem(ci, 3)
      for j in range(KJ):
        pltpu.async_copy(rows_v.at[p2, j], acc.at[dst_v.at[ph, j]], ssem,
                         add=True)

    def scatter_wait(ci):
      p2 = lax.rem(ci, 2)
      for j in range(KJ):
        pltpu.make_async_copy(rows_v.at[p2, j], acc.at[pl.ds(0, KB)],
                              ssem).wait()

    c0 = jnp.int32(0)
    idx_issue(c0)
    idx_wait(c0)
    adjust(c0)
    gather_issue(c0)
    idx_issue(c0 + 1)

    def chunk(ci, _):
      @pl.when(ci + 1 < NCHUNK)
      def _():
        idx_wait(ci + 1)
        adjust(ci + 1)

      @pl.when(ci >= 1)
      def _():
        scatter_wait(ci - 1)

      @pl.when(ci + 1 < NCHUNK)
      def _():
        gather_issue(ci + 1)

      @pl.when(ci + 2 < NCHUNK)
      def _():
        idx_issue(ci + 2)

      gather_wait(ci)
      scale(ci)
      scatter_issue(ci)
      return 0

    lax.fori_loop(0, NCHUNK, chunk, 0)
    scatter_wait(NCHUNK - 1)

    plsc.subcore_barrier()

    def _copy_out(segs):
      base = sid * RPT
      sems = (gsem0, gsem1)

      def _wr_ref(n):
        o, s = segs[n]
        return (rows_v.at[0, n % 2, pl.ds(0, s)],
                avt.at[pl.ds(cid * D + base + o, s)])

      for n, (o, s) in enumerate(segs):
        if n >= 2:
          psrc, pdst = _wr_ref(n - 2)
          pltpu.make_async_copy(psrc, pdst, sems[n % 2]).wait()
        pltpu.sync_copy(acc.at[pl.ds(base + o, s)],
                        rows_v.at[0, n % 2, pl.ds(0, s)])
        src, dst = _wr_ref(n)
        pltpu.async_copy(src, dst, sems[n % 2])
      for n in range(max(0, len(segs) - 2), len(segs)):
        psrc, pdst = _wr_ref(n)
        pltpu.make_async_copy(psrc, pdst, sems[n % 2]).wait()

    @pl.when(sid < NS - 1)
    def _():
      _copy_out(SEGS_FULL)

    @pl.when(sid == NS - 1)
    def _():
      _copy_out(SEGS_LAST)

  return sc_scatter


def _make_sc_degree(D, E):
  EDIR = 2 * E
  NW = NC * NS
  ETW = EDIR // NW
  DKB = 40
  DROWS = ETW // DKB
  mesh = plsc.VectorSubcoreMesh(core_axis_name="c", subcore_axis_name="s")

  RPT = -(D // -NS) + 7 - (-(D // -NS) + 7) % 8
  RLAST = D - (NS - 1) * RPT
  ZR = RPT + (LANES - RPT % LANES) % LANES

  @functools.partial(
      pl.kernel,
      out_type=jax.ShapeDtypeStruct((NC * D,), jnp.float32),
      mesh=mesh,
      compiler_params=pltpu.CompilerParams(use_tc_tiling_on_sc=False),
      scratch_types=[
          pltpu.VMEM((DROWS, DKB), jnp.int32),
          pltpu.VMEM((DROWS, DKB), jnp.float32),
          pltpu.VMEM((ZR,), jnp.float32),
          pltpu.VMEM((D,), jnp.float32),
          pltpu.VMEM_SHARED((D,), jnp.float32),
      ],
  )
  def sc_degree(dst2, w2, degp, dst_v, w_v, zbuf, stage_v, dacc):
    cid = lax.axis_index("c")
    sid = lax.axis_index("s")
    wid = sid * NC + cid
    zero16 = jnp.zeros((LANES,), jnp.float32)

    def _z(g, _):
      zbuf[pl.ds(g * LANES, LANES)] = zero16
      return 0
    lax.fori_loop(0, ZR // LANES, _z, 0)

    @pl.when(sid < NS - 1)
    def _():
      pltpu.sync_copy(zbuf.at[pl.ds(0, RPT)], dacc.at[pl.ds(sid * RPT, RPT)])

    @pl.when(sid == NS - 1)
    def _():
      pltpu.sync_copy(zbuf.at[pl.ds(0, RLAST)],
                      dacc.at[pl.ds((NS - 1) * RPT, RLAST)])

    rbase = wid * DROWS
    wrbase = rbase - jnp.where(wid >= NW // 2, E // DKB, 0)
    pltpu.sync_copy(dst2.at[pl.ds(rbase, DROWS)], dst_v)
    pltpu.sync_copy(w2.at[pl.ds(wrbase, DROWS)], w_v)

    plsc.subcore_barrier()

    def scat(j, _):
      pltpu.sync_copy(w_v.at[j], dacc.at[dst_v.at[j]], add=True)
      return 0
    lax.fori_loop(0, DROWS, scat, 0)

    plsc.subcore_barrier()

    @pl.when(sid == 0)
    def _():
      pltpu.sync_copy(dacc, stage_v)
      pltpu.sync_copy(stage_v, degp.at[pl.ds(cid * D, D)])

  return sc_degree


def kernel(v, edge_weights, edge_index):
  B, D = v.shape
  E = edge_index.shape[1]
  ei = edge_index.astype(jnp.int32)
  dsta = ei.reshape(2 * E)
  srca = jnp.flip(ei, axis=0).reshape(2 * E)
  table3, w2 = _prep(v, edge_weights.reshape(8, E // 8), D)
  sc = _make_sc_scatter(D, E)
  avt = sc(table3.reshape(2 * D, 32), w2.reshape(E), dsta, srca)
  scd = _make_sc_degree(D, E)
  degp = scd(dsta.reshape(-1, 40), w2.reshape(-1, 40))
  return _combine(v, avt.reshape(2, D, 32), degp.reshape(NC, D), D)


# speedup vs baseline: 3.1367x; 3.1367x over previous
import functools

import jax
import jax.numpy as jnp
from jax import lax
from jax.experimental import pallas as pl
from jax.experimental.pallas import tpu as pltpu
from jax.experimental.pallas import tpu_sc as plsc

NC = 2
NS = 16
LANES = 16

KB = 80
KJ = 5
K = KB * KJ


def _prep(v, ew2, D):
  db = 5120
  nblk = -(D // -db)
  R, C = ew2.shape

  def body(v_ref, ew_ref, ot_ref, ow_ref):
    ot_ref[...] = v_ref[...].T[None]
    x = ew_ref[...]
    ow_ref[...] = jnp.maximum(x, 0.0) + jnp.log(1.0 + jnp.exp(-jnp.abs(x)))

  return pl.pallas_call(
      body,
      grid=(2, nblk),
      in_specs=[
          pl.BlockSpec((32, db), lambda c, i: (c, i)),
          pl.BlockSpec((R, db), lambda c, i: (0, i)),
      ],
      out_specs=[
          pl.BlockSpec((1, db, 32), lambda c, i: (c, i, 0)),
          pl.BlockSpec((R, db), lambda c, i: (0, i)),
      ],
      out_shape=[
          jax.ShapeDtypeStruct((2, D, 32), jnp.float32),
          jax.ShapeDtypeStruct((R, C), jnp.float32),
      ],
  )(v, ew2)


def _combine(v, avt3, degp, D):
  db = 5120
  nblk = -(D // -db)

  def body(v_ref, a_ref, d_ref, o_ref):
    deg = jnp.sum(d_ref[...], axis=0)
    av = a_ref[0].T
    o_ref[...] = deg[None, :] * v_ref[...] - av

  return pl.pallas_call(
      body,
      grid=(2, nblk),
      in_specs=[
          pl.BlockSpec((32, db), lambda c, i: (c, i)),
          pl.BlockSpec((1, db, 32), lambda c, i: (c, i, 0)),
          pl.BlockSpec((NC, db), lambda c, i: (0, i)),
      ],
      out_specs=pl.BlockSpec((32, db), lambda c, i: (c, i)),
      out_shape=jax.ShapeDtypeStruct((64, D), jnp.float32),
  )(v, avt3, degp)


def _make_sc_scatter(D, E):
  EDIR = 2 * E
  EPT = EDIR // NS
  NCHUNK = EPT // K
  RPT = -(D // -NS) + 7 - (-(D // -NS) + 7) % 8
  RLAST = D - (NS - 1) * RPT
  mesh = plsc.VectorSubcoreMesh(core_axis_name="c", subcore_axis_name="s")

  def _segs(total):
    out, o = [], 0
    while o < total:
      s = min(KB, total - o)
      out.append((o, s))
      o += s
    return out

  SEGS_FULL = _segs(RPT)
  SEGS_LAST = _segs(RLAST)

  @functools.partial(
      pl.kernel,
      out_type=jax.ShapeDtypeStruct((2 * D, 32), jnp.float32),
      mesh=mesh,
      compiler_params=pltpu.CompilerParams(use_tc_tiling_on_sc=False),
      scratch_types=[
          pltpu.VMEM((3, KJ, KB), jnp.int32),
          pltpu.VMEM((3, KJ, KB), jnp.int32),
          pltpu.VMEM((3, K), jnp.float32),
          pltpu.VMEM((2, KJ, KB, 32), jnp.float32),
          pltpu.VMEM_SHARED((D, 32), jnp.float32),
          pltpu.SemaphoreType.DMA,
          pltpu.SemaphoreType.DMA,
          pltpu.SemaphoreType.DMA,
          pltpu.SemaphoreType.DMA,
      ],
  )
  def sc_scatter(table, wvec, r0, r1, avt,
                 dst_v, src_v, w_v, rows_v, acc,
                 lsem, gsem0, gsem1, ssem):
    cid = lax.axis_index("c")
    sid = lax.axis_index("s")
    zero16 = jnp.zeros((LANES,), jnp.float32)

    def _z2(r, _):
      rows_v[0, 0, r, pl.ds(0, LANES)] = zero16
      rows_v[0, 0, r, pl.ds(LANES, LANES)] = zero16
      return 0
    lax.fori_loop(0, KB, _z2, 0)

    def _zero_acc(segs):
      base = sid * RPT
      for (o, s) in segs:
        pltpu.sync_copy(rows_v.at[0, 0, pl.ds(0, s)],
                        acc.at[pl.ds(base + o, s)])

    @pl.when(sid < NS - 1)
    def _():
      _zero_acc(SEGS_FULL)

    @pl.when(sid == NS - 1)
    def _():
      _zero_acc(SEGS_LAST)

    plsc.subcore_barrier()

    coff16 = jnp.full((LANES,), cid * D, jnp.int32)
    ebase = sid * EPT
    wbase = ebase - jnp.where(sid >= NS // 2, E, 0)

    def idx_issue(ci):
      ph = lax.rem(ci, 3)
      for half, dref, sref in ((0, r0, r1), (1, r1, r0)):
        @pl.when((sid >= NS // 2) == (half == 1))
        def _(dref=dref, sref=sref, half=half):
          off = ebase - half * E + ci * K
          for j in range(KJ):
            pltpu.async_copy(dref.at[pl.ds(off + j * KB, KB)],
                             dst_v.at[ph, j], lsem)
            pltpu.async_copy(sref.at[pl.ds(off + j * KB, KB)],
                             src_v.at[ph, j], lsem)
      pltpu.async_copy(wvec.at[pl.ds(wbase + ci * K, K)], w_v.at[ph], lsem)

    def idx_wait(ci):
      ph = lax.rem(ci, 3)
      for j in range(KJ):
        pltpu.make_async_copy(r0.at[pl.ds(0, KB)], dst_v.at[ph, j],
                              lsem).wait()
        pltpu.make_async_copy(r0.at[pl.ds(0, KB)], src_v.at[ph, j],
                              lsem).wait()
      pltpu.make_async_copy(wvec.at[pl.ds(0, K)], w_v.at[ph], lsem).wait()

    def adjust(ci):
      ph = lax.rem(ci, 3)
      for j in range(KJ):
        for g in range(KB // LANES):
          sl = pl.ds(g * LANES, LANES)
          src_v[ph, j, sl] = src_v[ph, j, sl] + coff16

    def gather_issue(ci):
      ph = lax.rem(ci, 3)
      for p2, sem in ((0, gsem0), (1, gsem1)):
        @pl.when(lax.rem(ci, 2) == p2)
        def _(p2=p2, sem=sem):
          for j in range(KJ):
            pltpu.async_copy(table.at[src_v.at[ph, j]], rows_v.at[p2, j], sem)

    def gather_wait(ci):
      for p2, sem in ((0, gsem0), (1, gsem1)):
        @pl.when(lax.rem(ci, 2) == p2)
        def _(p2=p2, sem=sem):
          for j in range(KJ):
            pltpu.make_async_copy(table.at[pl.ds(0, KB)], rows_v.at[p2, j],
                                  sem).wait()

    def scale(ci):
      p2 = lax.rem(ci, 2)
      ph = lax.rem(ci, 3)
      for j in range(KJ):
        def body(g, _):
          w16 = w_v[ph, pl.ds(j * KB + g * LANES, LANES)]
          for i in range(LANES):
            r = g * LANES + i
            ws = jnp.full((LANES,), w16[i], jnp.float32)
            rows_v[p2, j, r, pl.ds(0, LANES)] = (
                rows_v[p2, j, r, pl.ds(0, LANES)] * ws)
            rows_v[p2, j, r, pl.ds(LANES, LANES)] = (
                rows_v[p2, j, r, pl.ds(LANES, LANES)] * ws)
          return 0
        lax.fori_loop(0, KB // LANES, body, 0)

    def scatter_issue(ci):
      p2 = lax.rem(ci, 2)
      ph = lax.rem(ci, 3)
      for j in range(KJ):
        pltpu.async_copy(rows_v.at[p2, j], acc.at[dst_v.at[ph, j]], ssem,
                         add=True)

    def scatter_wait(ci):
      p2 = lax.rem(ci, 2)
      for j in range(KJ):
        pltpu.make_async_copy(rows_v.at[p2, j], acc.at[pl.ds(0, KB)],
                              ssem).wait()

    c0 = jnp.int32(0)
    idx_issue(c0)
    idx_wait(c0)
    adjust(c0)
    gather_issue(c0)
    idx_issue(c0 + 1)

    def chunk(ci, _):
      @pl.when(ci + 1 < NCHUNK)
      def _():
        idx_wait(ci + 1)
        adjust(ci + 1)

      @pl.when(ci >= 1)
      def _():
        scatter_wait(ci - 1)

      @pl.when(ci + 1 < NCHUNK)
      def _():
        gather_issue(ci + 1)

      @pl.when(ci + 2 < NCHUNK)
      def _():
        idx_issue(ci + 2)

      gather_wait(ci)
      scale(ci)
      scatter_issue(ci)
      return 0

    lax.fori_loop(0, NCHUNK, chunk, 0)
    scatter_wait(NCHUNK - 1)

    plsc.subcore_barrier()

    def _copy_out(segs):
      base = sid * RPT
      sems = (gsem0, gsem1)

      def _wr_ref(n):
        o, s = segs[n]
        return (rows_v.at[0, n % 2, pl.ds(0, s)],
                avt.at[pl.ds(cid * D + base + o, s)])

      for n, (o, s) in enumerate(segs):
        if n >= 2:
          psrc, pdst = _wr_ref(n - 2)
          pltpu.make_async_copy(psrc, pdst, sems[n % 2]).wait()
        pltpu.sync_copy(acc.at[pl.ds(base + o, s)],
                        rows_v.at[0, n % 2, pl.ds(0, s)])
        src, dst = _wr_ref(n)
        pltpu.async_copy(src, dst, sems[n % 2])
      for n in range(max(0, len(segs) - 2), len(segs)):
        psrc, pdst = _wr_ref(n)
        pltpu.make_async_copy(psrc, pdst, sems[n % 2]).wait()

    @pl.when(sid < NS - 1)
    def _():
      _copy_out(SEGS_FULL)

    @pl.when(sid == NS - 1)
    def _():
      _copy_out(SEGS_LAST)

  return sc_scatter


def _make_sc_degree(D, E):
  EDIR = 2 * E
  NW = NC * NS
  ETW = EDIR // NW
  DKB = 40
  DROWS = ETW // DKB
  mesh = plsc.VectorSubcoreMesh(core_axis_name="c", subcore_axis_name="s")

  RPT = -(D // -NS) + 7 - (-(D // -NS) + 7) % 8
  RLAST = D - (NS - 1) * RPT
  ZR = RPT + (LANES - RPT % LANES) % LANES

  @functools.partial(
      pl.kernel,
      out_type=jax.ShapeDtypeStruct((NC * D,), jnp.float32),
      mesh=mesh,
      compiler_params=pltpu.CompilerParams(use_tc_tiling_on_sc=False),
      scratch_types=[
          pltpu.VMEM((DROWS, DKB), jnp.int32),
          pltpu.VMEM((DROWS, DKB), jnp.float32),
          pltpu.VMEM((ZR,), jnp.float32),
          pltpu.VMEM((D,), jnp.float32),
          pltpu.VMEM_SHARED((D,), jnp.float32),
      ],
  )
  def sc_degree(dst0, dst1, w2, degp, dst_v, w_v, zbuf, stage_v, dacc):
    cid = lax.axis_index("c")
    sid = lax.axis_index("s")
    wid = sid * NC + cid
    zero16 = jnp.zeros((LANES,), jnp.float32)

    def _z(g, _):
      zbuf[pl.ds(g * LANES, LANES)] = zero16
      return 0
    lax.fori_loop(0, ZR // LANES, _z, 0)

    @pl.when(sid < NS - 1)
    def _():
      pltpu.sync_copy(zbuf.at[pl.ds(0, RPT)], dacc.at[pl.ds(sid * RPT, RPT)])

    @pl.when(sid == NS - 1)
    def _():
      pltpu.sync_copy(zbuf.at[pl.ds(0, RLAST)],
                      dacc.at[pl.ds((NS - 1) * RPT, RLAST)])

    rbase = wid * DROWS
    wrbase = rbase - jnp.where(wid >= NW // 2, E // DKB, 0)

    @pl.when(wid < NW // 2)
    def _():
      pltpu.sync_copy(dst0.at[pl.ds(rbase, DROWS)], dst_v)

    @pl.when(wid >= NW // 2)
    def _():
      pltpu.sync_copy(dst1.at[pl.ds(rbase - E // DKB, DROWS)], dst_v)

    pltpu.sync_copy(w2.at[pl.ds(wrbase, DROWS)], w_v)

    plsc.subcore_barrier()

    def scat(j, _):
      pltpu.sync_copy(w_v.at[j], dacc.at[dst_v.at[j]], add=True)
      return 0
    lax.fori_loop(0, DROWS, scat, 0)

    plsc.subcore_barrier()

    @pl.when(sid == 0)
    def _():
      pltpu.sync_copy(dacc, stage_v)
      pltpu.sync_copy(stage_v, degp.at[pl.ds(cid * D, D)])

  return sc_degree


def kernel(v, edge_weights, edge_index):
  B, D = v.shape
  E = edge_index.shape[1]
  ei = edge_index.astype(jnp.int32)
  r0 = ei[0]
  r1 = ei[1]
  table3, w2 = _prep(v, edge_weights.reshape(8, E // 8), D)
  sc = _make_sc_scatter(D, E)
  avt = sc(table3.reshape(2 * D, 32), w2.reshape(E), r0, r1)
  scd = _make_sc_degree(D, E)
  degp = scd(r0.reshape(-1, 40), r1.reshape(-1, 40), w2.reshape(-1, 40))
  return _combine(v, avt.reshape(2, D, 32), degp.reshape(NC, D), D)


# speedup vs baseline: 3.1905x; 1.0171x over previous
import functools

import jax
import jax.numpy as jnp
from jax import lax
from jax.experimental import pallas as pl
from jax.experimental.pallas import tpu as pltpu
from jax.experimental.pallas import tpu_sc as plsc

NC = 2
NS = 16
LANES = 16

KB = 80
KJ = 5
K = KB * KJ


def _prep(v, ew2, D):
  db = 5120
  nblk = -(D // -db)
  R, C = ew2.shape

  def body(v_ref, ew_ref, ot_ref, ow_ref):
    ot_ref[...] = v_ref[...].T[None]
    x = ew_ref[...]
    ow_ref[...] = jnp.maximum(x, 0.0) + jnp.log(1.0 + jnp.exp(-jnp.abs(x)))

  return pl.pallas_call(
      body,
      grid=(2, nblk),
      in_specs=[
          pl.BlockSpec((32, db), lambda c, i: (c, i)),
          pl.BlockSpec((R, db), lambda c, i: (0, i)),
      ],
      out_specs=[
          pl.BlockSpec((1, db, 32), lambda c, i: (c, i, 0)),
          pl.BlockSpec((R, db), lambda c, i: (0, i)),
      ],
      out_shape=[
          jax.ShapeDtypeStruct((2, D, 32), jnp.float32),
          jax.ShapeDtypeStruct((R, C), jnp.float32),
      ],
  )(v, ew2)


def _combine(v, avt3, degp, D):
  db = 5120
  nblk = -(D // -db)

  def body(v_ref, a_ref, d_ref, o_ref):
    deg = jnp.sum(d_ref[...], axis=0)
    av = a_ref[0].T
    o_ref[...] = deg[None, :] * v_ref[...] - av

  return pl.pallas_call(
      body,
      grid=(2, nblk),
      in_specs=[
          pl.BlockSpec((32, db), lambda c, i: (c, i)),
          pl.BlockSpec((1, db, 32), lambda c, i: (c, i, 0)),
          pl.BlockSpec((NC, db), lambda c, i: (0, i)),
      ],
      out_specs=pl.BlockSpec((32, db), lambda c, i: (c, i)),
      out_shape=jax.ShapeDtypeStruct((64, D), jnp.float32),
  )(v, avt3, degp)


def _make_sc_scatter(D, E):
  EDIR = 2 * E
  EPT = EDIR // NS
  NCHUNK = EPT // K
  RPT = -(D // -NS) + 7 - (-(D // -NS) + 7) % 8
  RLAST = D - (NS - 1) * RPT
  mesh = plsc.VectorSubcoreMesh(core_axis_name="c", subcore_axis_name="s")

  def _segs(total):
    out, o = [], 0
    while o < total:
      s = min(KB, total - o)
      out.append((o, s))
      o += s
    return out

  SEGS_FULL = _segs(RPT)
  SEGS_LAST = _segs(RLAST)

  @functools.partial(
      pl.kernel,
      out_type=jax.ShapeDtypeStruct((2 * D, 32), jnp.float32),
      mesh=mesh,
      compiler_params=pltpu.CompilerParams(use_tc_tiling_on_sc=False),
      scratch_types=[
          pltpu.VMEM((3, KJ, KB), jnp.int32),
          pltpu.VMEM((3, KJ, KB), jnp.int32),
          pltpu.VMEM((3, K), jnp.float32),
          pltpu.VMEM((2, K, 32), jnp.float32),
          pltpu.VMEM_SHARED((D, 32), jnp.float32),
          pltpu.SemaphoreType.DMA,
          pltpu.SemaphoreType.DMA,
          pltpu.SemaphoreType.DMA,
          pltpu.SemaphoreType.DMA,
      ],
  )
  def sc_scatter(table, wvec, r0, r1, avt,
                 dst_v, src_v, w_v, rows_v, acc,
                 lsem, gsem0, gsem1, ssem):
    cid = lax.axis_index("c")
    sid = lax.axis_index("s")
    zero16 = jnp.zeros((LANES,), jnp.float32)

    def _z2(r, _):
      rows_v[0, r, pl.ds(0, LANES)] = zero16
      rows_v[0, r, pl.ds(LANES, LANES)] = zero16
      return 0
    lax.fori_loop(0, KB, _z2, 0)

    def _zero_acc(segs):
      base = sid * RPT
      for (o, s) in segs:
        pltpu.sync_copy(rows_v.at[0, pl.ds(0, s)],
                        acc.at[pl.ds(base + o, s)])

    @pl.when(sid < NS - 1)
    def _():
      _zero_acc(SEGS_FULL)

    @pl.when(sid == NS - 1)
    def _():
      _zero_acc(SEGS_LAST)

    plsc.subcore_barrier()

    coff16 = jnp.full((LANES,), cid * D, jnp.int32)
    ebase = sid * EPT
    wbase = ebase - jnp.where(sid >= NS // 2, E, 0)

    def idx_issue(ci):
      ph = lax.rem(ci, 3)
      for half, dref, sref in ((0, r0, r1), (1, r1, r0)):
        @pl.when((sid >= NS // 2) == (half == 1))
        def _(dref=dref, sref=sref, half=half):
          off = ebase - half * E + ci * K
          for j in range(KJ):
            pltpu.async_copy(dref.at[pl.ds(off + j * KB, KB)],
                             dst_v.at[ph, j], lsem)
            pltpu.async_copy(sref.at[pl.ds(off + j * KB, KB)],
                             src_v.at[ph, j], lsem)
      pltpu.async_copy(wvec.at[pl.ds(wbase + ci * K, K)], w_v.at[ph], lsem)

    def idx_wait(ci):
      ph = lax.rem(ci, 3)
      for j in range(KJ):
        pltpu.make_async_copy(r0.at[pl.ds(0, KB)], dst_v.at[ph, j],
                              lsem).wait()
        pltpu.make_async_copy(r0.at[pl.ds(0, KB)], src_v.at[ph, j],
                              lsem).wait()
      pltpu.make_async_copy(wvec.at[pl.ds(0, K)], w_v.at[ph], lsem).wait()

    def adjust(ci):
      ph = lax.rem(ci, 3)
      for j in range(KJ):
        for g in range(KB // LANES):
          sl = pl.ds(g * LANES, LANES)
          src_v[ph, j, sl] = src_v[ph, j, sl] + coff16

    def gather_issue(ci):
      ph = lax.rem(ci, 3)
      for p2, sem in ((0, gsem0), (1, gsem1)):
        @pl.when(lax.rem(ci, 2) == p2)
        def _(p2=p2, sem=sem):
          for j in range(KJ):
            pltpu.async_copy(table.at[src_v.at[ph, j]],
                             rows_v.at[p2, pl.ds(j * KB, KB)], sem)

    def gather_wait(ci):
      for p2, sem in ((0, gsem0), (1, gsem1)):
        @pl.when(lax.rem(ci, 2) == p2)
        def _(p2=p2, sem=sem):
          for j in range(KJ):
            pltpu.make_async_copy(table.at[pl.ds(0, KB)],
                                  rows_v.at[p2, pl.ds(j * KB, KB)],
                                  sem).wait()

    def scale(ci):
      p2 = lax.rem(ci, 2)
      ph = lax.rem(ci, 3)

      @plsc.parallel_loop(0, K // LANES, unroll=2)
      def _(g):
        w16 = w_v[ph, pl.ds(g * LANES, LANES)]
        for i in range(LANES):
          r = g * LANES + i
          ws = jnp.full((LANES,), w16[i], jnp.float32)
          rows_v[p2, r, pl.ds(0, LANES)] = (
              rows_v[p2, r, pl.ds(0, LANES)] * ws)
          rows_v[p2, r, pl.ds(LANES, LANES)] = (
              rows_v[p2, r, pl.ds(LANES, LANES)] * ws)

    def scatter_issue(ci):
      p2 = lax.rem(ci, 2)
      ph = lax.rem(ci, 3)
      for j in range(KJ):
        pltpu.async_copy(rows_v.at[p2, pl.ds(j * KB, KB)],
                         acc.at[dst_v.at[ph, j]], ssem, add=True)

    def scatter_wait(ci):
      p2 = lax.rem(ci, 2)
      for j in range(KJ):
        pltpu.make_async_copy(rows_v.at[p2, pl.ds(j * KB, KB)],
                              acc.at[pl.ds(0, KB)], ssem).wait()

    c0 = jnp.int32(0)
    idx_issue(c0)
    idx_wait(c0)
    adjust(c0)
    gather_issue(c0)
    idx_issue(c0 + 1)

    def chunk(ci, _):
      @pl.when(ci + 1 < NCHUNK)
      def _():
        idx_wait(ci + 1)
        adjust(ci + 1)

      @pl.when(ci >= 1)
      def _():
        scatter_wait(ci - 1)

      @pl.when(ci + 1 < NCHUNK)
      def _():
        gather_issue(ci + 1)

      @pl.when(ci + 2 < NCHUNK)
      def _():
        idx_issue(ci + 2)

      gather_wait(ci)
      scale(ci)
      scatter_issue(ci)
      return 0

    lax.fori_loop(0, NCHUNK, chunk, 0)
    scatter_wait(NCHUNK - 1)

    plsc.subcore_barrier()

    def _copy_out(segs):
      base = sid * RPT
      sems = (gsem0, gsem1)

      def _wr_ref(n):
        o, s = segs[n]
        return (rows_v.at[n % 2, pl.ds(0, s)],
                avt.at[pl.ds(cid * D + base + o, s)])

      for n, (o, s) in enumerate(segs):
        if n >= 2:
          psrc, pdst = _wr_ref(n - 2)
          pltpu.make_async_copy(psrc, pdst, sems[n % 2]).wait()
        pltpu.sync_copy(acc.at[pl.ds(base + o, s)],
                        rows_v.at[n % 2, pl.ds(0, s)])
        src, dst = _wr_ref(n)
        pltpu.async_copy(src, dst, sems[n % 2])
      for n in range(max(0, len(segs) - 2), len(segs)):
        psrc, pdst = _wr_ref(n)
        pltpu.make_async_copy(psrc, pdst, sems[n % 2]).wait()

    @pl.when(sid < NS - 1)
    def _():
      _copy_out(SEGS_FULL)

    @pl.when(sid == NS - 1)
    def _():
      _copy_out(SEGS_LAST)

  return sc_scatter


def _make_sc_degree(D, E):
  EDIR = 2 * E
  NW = NC * NS
  ETW = EDIR // NW
  DKB = 40
  DROWS = ETW // DKB
  mesh = plsc.VectorSubcoreMesh(core_axis_name="c", subcore_axis_name="s")

  RPT = -(D // -NS) + 7 - (-(D // -NS) + 7) % 8
  RLAST = D - (NS - 1) * RPT
  ZR = RPT + (LANES - RPT % LANES) % LANES

  @functools.partial(
      pl.kernel,
      out_type=jax.ShapeDtypeStruct((NC * D,), jnp.float32),
      mesh=mesh,
      compiler_params=pltpu.CompilerParams(use_tc_tiling_on_sc=False),
      scratch_types=[
          pltpu.VMEM((DROWS, DKB), jnp.int32),
          pltpu.VMEM((DROWS, DKB), jnp.float32),
          pltpu.VMEM((ZR,), jnp.float32),
          pltpu.VMEM((D,), jnp.float32),
          pltpu.VMEM_SHARED((D,), jnp.float32),
      ],
  )
  def sc_degree(dst0, dst1, w2, degp, dst_v, w_v, zbuf, stage_v, dacc):
    cid = lax.axis_index("c")
    sid = lax.axis_index("s")
    wid = sid * NC + cid
    zero16 = jnp.zeros((LANES,), jnp.float32)

    def _z(g, _):
      zbuf[pl.ds(g * LANES, LANES)] = zero16
      return 0
    lax.fori_loop(0, ZR // LANES, _z, 0)

    @pl.when(sid < NS - 1)
    def _():
      pltpu.sync_copy(zbuf.at[pl.ds(0, RPT)], dacc.at[pl.ds(sid * RPT, RPT)])

    @pl.when(sid == NS - 1)
    def _():
      pltpu.sync_copy(zbuf.at[pl.ds(0, RLAST)],
                      dacc.at[pl.ds((NS - 1) * RPT, RLAST)])

    rbase = wid * DROWS
    wrbase = rbase - jnp.where(wid >= NW // 2, E // DKB, 0)

    @pl.when(wid < NW // 2)
    def _():
      pltpu.sync_copy(dst0.at[pl.ds(rbase, DROWS)], dst_v)

    @pl.when(wid >= NW // 2)
    def _():
      pltpu.sync_copy(dst1.at[pl.ds(rbase - E // DKB, DROWS)], dst_v)

    pltpu.sync_copy(w2.at[pl.ds(wrbase, DROWS)], w_v)

    plsc.subcore_barrier()

    def scat(j, _):
      pltpu.sync_copy(w_v.at[j], dacc.at[dst_v.at[j]], add=True)
      return 0
    lax.fori_loop(0, DROWS, scat, 0)

    plsc.subcore_barrier()

    @pl.when(sid == 0)
    def _():
      pltpu.sync_copy(dacc, stage_v)
      pltpu.sync_copy(stage_v, degp.at[pl.ds(cid * D, D)])

  return sc_degree


def kernel(v, edge_weights, edge_index):
  B, D = v.shape
  E = edge_index.shape[1]
  ei = edge_index.astype(jnp.int32)
  r0 = ei[0]
  r1 = ei[1]
  table3, w2 = _prep(v, edge_weights.reshape(8, E // 8), D)
  sc = _make_sc_scatter(D, E)
  avt = sc(table3.reshape(2 * D, 32), w2.reshape(E), r0, r1)
  scd = _make_sc_degree(D, E)
  degp = scd(r0.reshape(-1, 40), r1.reshape(-1, 40), w2.reshape(-1, 40))
  return _combine(v, avt.reshape(2, D, 32), degp.reshape(NC, D), D)
